# Initial kernel scaffold; baseline (speedup 1.0000x reference)
import jax
import jax.numpy as jnp
from jax.experimental import pallas as pl


def kernel(x, edge_index, adj_vals, W, b):
    raise NotImplementedError("write your pallas kernel here")


# speedup vs baseline: 9.1611x; 9.1611x over previous
import functools

import jax
import jax.numpy as jnp
from jax import lax
from jax.experimental import pallas as pl
from jax.experimental.pallas import tpu as pltpu
from jax.experimental.pallas import tpu_sc as plsc

N = 10000
E = 640000
D = 128
C = 128
NBLK = E // C
NW = 32
BASE_BLK = 152
EXTRA_W = 17
ZSPAN = 632
WB_BASE = 624


def _mm_body(x_ref, w_ref, o_ref):
    o_ref[...] = jnp.dot(x_ref[...], w_ref[...] * 0.5,
                         preferred_element_type=jnp.float32)


def _matmul_half(x, W):
    return pl.pallas_call(
        _mm_body,
        grid=(10,),
        in_specs=[
            pl.BlockSpec((1000, D), lambda i: (i, 0)),
            pl.BlockSpec((D, D), lambda i: (0, 0)),
        ],
        out_specs=pl.BlockSpec((1000, D), lambda i: (i, 0)),
        out_shape=jax.ShapeDtypeStruct((N, D), jnp.float32),
    )(x, W)


def _comb_body(p_ref, b_ref, o_ref):
    o_ref[...] = p_ref[0] + p_ref[1] + b_ref[...]


def _combine(partials, b2d):
    return pl.pallas_call(
        _comb_body,
        grid=(10,),
        in_specs=[
            pl.BlockSpec((2, 1000, D), lambda i: (0, i, 0)),
            pl.BlockSpec((1, D), lambda i: (0, 0)),
        ],
        out_specs=pl.BlockSpec((1000, D), lambda i: (i, 0)),
        out_shape=jax.ShapeDtypeStruct((N, D), jnp.float32),
    )(partials, b2d)


def _sc_scatter(support, dst2d, src2d, w2d):
    mesh = plsc.VectorSubcoreMesh(core_axis_name="c", subcore_axis_name="s")

    @functools.partial(
        pl.kernel,
        mesh=mesh,
        out_type=jax.ShapeDtypeStruct((2, N, D), jnp.float32),
        scratch_types=[
            pltpu.VMEM((8, C), jnp.int32),
            pltpu.VMEM((8, C), jnp.int32),
            pltpu.VMEM((8, C), jnp.float32),
            pltpu.VMEM((C, D), jnp.float32),
            pltpu.VMEM_SHARED((N, D), jnp.float32),
            pltpu.SemaphoreType.DMA,
        ],
    )
    def k(support_hbm, dst_hbm, src_hbm, w_hbm, out_hbm,
          dstb, srcb, wb, rows, acc, sem):
        cid = lax.axis_index("c")
        sid = lax.axis_index("s")
        wid = cid * 16 + sid
        n_oct = jnp.where(wid < EXTRA_W, BASE_BLK // 8 + 1, BASE_BLK // 8)
        b0 = wid * BASE_BLK + jnp.minimum(wid, EXTRA_W) * 8

        zero16 = jnp.zeros((16,), jnp.float32)

        def zbody(i, carry):
            for g in range(8):
                rows[i, pl.ds(g * 16, 16)] = zero16
            return carry

        lax.fori_loop(0, C, zbody, 0)
        zbase = (sid * 625) // 8 * 8
        for kk in range(4):
            pltpu.sync_copy(rows, acc.at[pl.ds(zbase + kk * C, C)])
        pltpu.sync_copy(rows.at[pl.ds(0, ZSPAN - 4 * C)],
                        acc.at[pl.ds(zbase + 4 * C, ZSPAN - 4 * C)])
        plsc.subcore_barrier()

        def octet(o, carry):
            ob = b0 + o * 8
            pltpu.sync_copy(dst_hbm.at[pl.ds(ob, 8)], dstb)
            pltpu.sync_copy(src_hbm.at[pl.ds(ob, 8)], srcb)
            pltpu.sync_copy(w_hbm.at[pl.ds(ob, 8)], wb)

            def chunk(j, c1):
                pltpu.async_copy(support_hbm.at[srcb.at[j]], rows, sem).wait()

                def scale(grp, c2):
                    wvec = wb[j, pl.ds(grp * 16, 16)]
                    for l in range(16):
                        e = grp * 16 + l
                        wv = lax.broadcast(wvec[l], (16,))
                        for g in range(8):
                            sl = pl.ds(g * 16, 16)
                            rows[e, sl] = rows[e, sl] * wv
                    return c2

                lax.fori_loop(0, C // 16, scale, 0)
                pltpu.sync_copy(rows, acc.at[dstb.at[j]], add=True)
                return c1

            lax.fori_loop(0, 8, chunk, 0)
            return carry

        lax.fori_loop(0, n_oct, octet, 0)
        plsc.subcore_barrier()

        wb_off = WB_BASE * sid + jnp.minimum(sid, 2) * 8

        @pl.when(sid < 2)
        def _():
            pltpu.sync_copy(acc.at[pl.ds(wb_off, WB_BASE + 8)],
                            out_hbm.at[cid, pl.ds(wb_off, WB_BASE + 8)])

        @pl.when(sid >= 2)
        def _():
            pltpu.sync_copy(acc.at[pl.ds(wb_off, WB_BASE)],
                            out_hbm.at[cid, pl.ds(wb_off, WB_BASE)])

    return k(support, dst2d, src2d, w2d)


def kernel(x, edge_index, adj_vals, W, b):
    support = _matmul_half(x, W)
    dst2d = edge_index[0].reshape(NBLK, C)
    src2d = edge_index[1].reshape(NBLK, C)
    w2d = adj_vals.reshape(NBLK, C)
    partials = _sc_scatter(support, dst2d, src2d, w2d)
    return _combine(partials, b.reshape(1, D))


# speedup vs baseline: 12.0160x; 1.3116x over previous
import functools

import jax
import jax.numpy as jnp
from jax import lax
from jax.experimental import pallas as pl
from jax.experimental.pallas import tpu as pltpu
from jax.experimental.pallas import tpu_sc as plsc

N = 10000
E = 640000
D = 128
C = 64
NBLK = E // C
NW = 32
BASE_BLK = 304
EXTRA_W = 17
ZSPAN = 632
WB_BASE = 624


def _mm_body(x_ref, w_ref, o_ref):
    o_ref[...] = jnp.dot(x_ref[...], w_ref[...] * 0.5,
                         preferred_element_type=jnp.float32)


def _matmul_half(x, W):
    return pl.pallas_call(
        _mm_body,
        grid=(10,),
        in_specs=[
            pl.BlockSpec((1000, D), lambda i: (i, 0)),
            pl.BlockSpec((D, D), lambda i: (0, 0)),
        ],
        out_specs=pl.BlockSpec((1000, D), lambda i: (i, 0)),
        out_shape=jax.ShapeDtypeStruct((N, D), jnp.float32),
    )(x, W)


def _comb_body(p_ref, b_ref, o_ref):
    o_ref[...] = p_ref[0] + p_ref[1] + b_ref[...]


def _combine(partials, b2d):
    return pl.pallas_call(
        _comb_body,
        grid=(10,),
        in_specs=[
            pl.BlockSpec((2, 1000, D), lambda i: (0, i, 0)),
            pl.BlockSpec((1, D), lambda i: (0, 0)),
        ],
        out_specs=pl.BlockSpec((1000, D), lambda i: (i, 0)),
        out_shape=jax.ShapeDtypeStruct((N, D), jnp.float32),
    )(partials, b2d)


def _sc_scatter(support, dst2d, src2d, w2d):
    mesh = plsc.VectorSubcoreMesh(core_axis_name="c", subcore_axis_name="s")

    @functools.partial(
        pl.kernel,
        mesh=mesh,
        out_type=jax.ShapeDtypeStruct((2, N, D), jnp.float32),
        scratch_types=[
            pltpu.VMEM((8, C), jnp.int32),
            pltpu.VMEM((8, C), jnp.int32),
            pltpu.VMEM((8, C), jnp.float32),
            pltpu.VMEM((C, D), jnp.float32),
            pltpu.VMEM((C, D), jnp.float32),
            pltpu.VMEM((C, D), jnp.float32),
            pltpu.VMEM((C, D), jnp.float32),
            pltpu.VMEM_SHARED((N, D), jnp.float32),
            pltpu.SemaphoreType.DMA,
            pltpu.SemaphoreType.DMA,
            pltpu.SemaphoreType.DMA,
            pltpu.SemaphoreType.DMA,
            pltpu.SemaphoreType.DMA,
            pltpu.SemaphoreType.DMA,
            pltpu.SemaphoreType.DMA,
            pltpu.SemaphoreType.DMA,
            pltpu.SemaphoreType.DMA,
        ],
    )
    def k(support_hbm, dst_hbm, src_hbm, w_hbm, out_hbm,
          dstb, srcb, wb, r0, r1, r2, r3, acc,
          g0, g1, g2, g3, s0, s1, s2, s3, isem):
        rows = [r0, r1, r2, r3]
        gsem = [g0, g1, g2, g3]
        ssem = [s0, s1, s2, s3]
        cid = lax.axis_index("c")
        sid = lax.axis_index("s")
        wid = cid * 16 + sid
        n_oct = jnp.where(wid < EXTRA_W, (BASE_BLK + 16) // 8, BASE_BLK // 8)
        b0 = wid * BASE_BLK + jnp.minimum(wid, EXTRA_W) * 16

        zero16 = jnp.zeros((16,), jnp.float32)

        def zbody(i, carry):
            for g in range(8):
                r0[i, pl.ds(g * 16, 16)] = zero16
            return carry

        lax.fori_loop(0, C, zbody, 0)
        zbase = (sid * 625) // 8 * 8
        for kk in range(9):
            pltpu.sync_copy(r0, acc.at[pl.ds(zbase + kk * C, C)])
        pltpu.sync_copy(r0.at[pl.ds(0, ZSPAN - 9 * C)],
                        acc.at[pl.ds(zbase + 9 * C, ZSPAN - 9 * C)])
        plsc.subcore_barrier()

        def scale(j, p):
            rp = rows[p]

            def body(gq, c2):
                wvec = wb[j, pl.ds(gq * 16, 16)]
                for l in range(16):
                    e = gq * 16 + l
                    wv = lax.broadcast(wvec[l], (16,))
                    for gg in range(8):
                        sl = pl.ds(gg * 16, 16)
                        rp[e, sl] = rp[e, sl] * wv
                return c2

            lax.fori_loop(0, C // 16, body, 0)

        def octet(o, carry):
            ob = b0 + o * 8
            i1 = pltpu.async_copy(dst_hbm.at[pl.ds(ob, 8)], dstb, isem)
            i2 = pltpu.async_copy(src_hbm.at[pl.ds(ob, 8)], srcb, isem)
            i3 = pltpu.async_copy(w_hbm.at[pl.ds(ob, 8)], wb, isem)
            i1.wait()
            i2.wait()
            i3.wait()
            g = {}
            s = {}
            for j in range(3):
                g[j] = pltpu.async_copy(support_hbm.at[srcb.at[j]],
                                        rows[j], gsem[j])
            for j in range(8):
                p = j % 4
                if j + 3 < 8:
                    q = (j + 3) % 4
                    if j >= 1:
                        s[j - 1].wait()
                    g[j + 3] = pltpu.async_copy(support_hbm.at[srcb.at[j + 3]],
                                                rows[q], gsem[q])
                g[j].wait()
                scale(j, p)
                s[j] = pltpu.async_copy(rows[p], acc.at[dstb.at[j]],
                                        ssem[p], add=True)
            for j in (4, 5, 6, 7):
                s[j].wait()
            return carry

        lax.fori_loop(0, n_oct, octet, 0)
        plsc.subcore_barrier()

        wb_off = WB_BASE * sid + jnp.minimum(sid, 2) * 8

        @pl.when(sid < 2)
        def _():
            pltpu.sync_copy(acc.at[pl.ds(wb_off, WB_BASE + 8)],
                            out_hbm.at[cid, pl.ds(wb_off, WB_BASE + 8)])

        @pl.when(sid >= 2)
        def _():
            pltpu.sync_copy(acc.at[pl.ds(wb_off, WB_BASE)],
                            out_hbm.at[cid, pl.ds(wb_off, WB_BASE)])

    return k(support, dst2d, src2d, w2d)


def kernel(x, edge_index, adj_vals, W, b):
    support = _matmul_half(x, W)
    dst2d = edge_index[0].reshape(NBLK, C)
    src2d = edge_index[1].reshape(NBLK, C)
    w2d = adj_vals.reshape(NBLK, C)
    partials = _sc_scatter(support, dst2d, src2d, w2d)
    return _combine(partials, b.reshape(1, D))
